# baseline (device time: 39407 ns/iter reference)
import jax
import jax.numpy as jnp
from jax import lax
from jax.experimental import pallas as pl
from jax.experimental.pallas import tpu as pltpu

H = 16
DH = 64
DR = 32
G = 4
HG = H // G
BF = jnp.bfloat16
F32 = jnp.float32
HC = HG * DH
RC = HG * DR


def _dot(a, b):
    return jnp.dot(a, b, preferred_element_type=F32)


def _dot_t(a, b):
    return lax.dot_general(a, b, (((1,), (1,)), ((), ())),
                           preferred_element_type=F32)


def kernel(x, Wdkv, Wuk, Wuv, Wq, Wqr, Wkr, Wo):
    B, S, D = x.shape
    dc = Wdkv.shape[1]
    scale = (DH + DR) ** -0.5

    def body(x_ref, wdkv_ref, wuk_ref, wuv_ref, wq_ref, wqr_ref, wkr_ref,
             wo_ref, out_ref,
             x16_ref, wukm_ref, wuvm_ref, kvps_ref, kvpr_ref,
             q_ref, qr_ref, kr_ref, wo16_ref, k16_ref, v16_ref, ofull_ref,
             kv_send_sem, kv_recv_sem, osend_sems, orecv_sems):
        my_x = lax.axis_index("x")
        my_y = lax.axis_index("y")
        my_z = lax.axis_index("z")
        partner = (1 - my_x, my_y, my_z)

        barrier = pltpu.get_barrier_semaphore()
        pl.semaphore_signal(barrier, inc=1, device_id=partner,
                            device_id_type=pl.DeviceIdType.MESH)
        for j in range(1, G):
            pl.semaphore_signal(barrier, inc=1,
                                device_id=(my_x, my_y, (my_z + j) % G),
                                device_id_type=pl.DeviceIdType.MESH)
        pl.semaphore_wait(barrier, G)

        for b in range(B):
            x16_ref[b] = x_ref[b].astype(BF)
        cs = [None] * B
        for b in range(B):
            cs[b] = _dot(x16_ref[b], wdkv_ref[...].astype(BF)).astype(BF)

        for g in range(G):
            @pl.when(my_z == g)
            def _(g=g):
                hc0 = g * HC
                wukm_ref[...] = wuk_ref[:, hc0:hc0 + HC].astype(BF)
                wuvm_ref[...] = wuv_ref[:, hc0:hc0 + HC].astype(BF)
        for b in range(B):
            kvps_ref[0, b] = _dot(cs[b], wukm_ref[...]).astype(BF)
            kvps_ref[1, b] = _dot(cs[b], wuvm_ref[...]).astype(BF)
        kv_rdma = pltpu.make_async_remote_copy(
            src_ref=kvps_ref, dst_ref=kvpr_ref,
            send_sem=kv_send_sem, recv_sem=kv_recv_sem,
            device_id=partner, device_id_type=pl.DeviceIdType.MESH)
        kv_rdma.start()

        for g in range(G):
            @pl.when(my_z == g)
            def _(g=g):
                hc0 = g * HC
                rc0 = g * RC
                wq16 = wq_ref[:, hc0:hc0 + HC].astype(BF)
                wqr16 = wqr_ref[:, rc0:rc0 + RC].astype(BF)
                for b in range(B):
                    q_ref[b] = (scale * _dot(x16_ref[b], wq16)).astype(BF)
                    qr_ref[b] = (scale * _dot(x16_ref[b], wqr16)).astype(BF)
        wkr16 = wkr_ref[...].astype(BF)
        for b in range(B):
            kr_ref[b] = _dot(x16_ref[b], wkr16).astype(BF)
        wo16_ref[...] = wo_ref[...].astype(BF)

        kv_rdma.wait()
        for b in range(B):
            k16_ref[b] = (kvps_ref[0, b] + kvpr_ref[0, b]).astype(BF)
            v16_ref[b] = (kvps_ref[1, b] + kvpr_ref[1, b]).astype(BF)

        o_rdmas = []
        for b in range(B):
            krb = kr_ref[b]
            for h in range(HG):
                qh = q_ref[b, :, h * DH:(h + 1) * DH]
                kh = k16_ref[b, :, h * DH:(h + 1) * DH]
                qrh = qr_ref[b, :, h * DR:(h + 1) * DR]
                s = _dot_t(qh, kh) + _dot_t(qrh, krb)
                p = jnp.exp(s)
                denom = jnp.sum(p, axis=-1, keepdims=True)
                oh = _dot(p.astype(BF), v16_ref[b, :, h * DH:(h + 1) * DH])
                ofull_ref[my_z, b, :, h * DH:(h + 1) * DH] = (
                    oh / denom).astype(BF)
            for j in range(1, G):
                rdma = pltpu.make_async_remote_copy(
                    src_ref=ofull_ref.at[my_z, b],
                    dst_ref=ofull_ref.at[my_z, b],
                    send_sem=osend_sems.at[(j - 1) * B + b],
                    recv_sem=orecv_sems.at[my_z, b],
                    device_id=(my_x, my_y, (my_z + j) % G),
                    device_id_type=pl.DeviceIdType.MESH)
                rdma.start()
                o_rdmas.append(rdma)

        for b in range(B):
            acc = None
            for g in range(G):
                @pl.when(my_z != g)
                def _(g=g, b=b):
                    recv = pltpu.make_async_remote_copy(
                        src_ref=ofull_ref.at[g, b],
                        dst_ref=ofull_ref.at[g, b],
                        send_sem=osend_sems.at[0],
                        recv_sem=orecv_sems.at[g, b],
                        device_id=partner,
                        device_id_type=pl.DeviceIdType.MESH)
                    recv.wait_recv()
                chunk = _dot(ofull_ref[g, b],
                             wo16_ref[g * HC:(g + 1) * HC, :])
                acc = chunk if acc is None else acc + chunk
            out_ref[b] = acc

        for rdma in o_rdmas:
            rdma.wait_send()

    return pl.pallas_call(
        body,
        out_shape=jax.ShapeDtypeStruct((B, S, D), F32),
        in_specs=[pl.BlockSpec(memory_space=pltpu.VMEM)] * 8,
        out_specs=pl.BlockSpec(memory_space=pltpu.VMEM),
        scratch_shapes=[
            pltpu.VMEM((B, S, D), BF),
            pltpu.VMEM((Wuk.shape[0], HC), BF),
            pltpu.VMEM((Wuv.shape[0], HC), BF),
            pltpu.VMEM((2, B, S, HC), BF),
            pltpu.VMEM((2, B, S, HC), BF),
            pltpu.VMEM((B, S, HC), BF),
            pltpu.VMEM((B, S, RC), BF),
            pltpu.VMEM((B, S, DR), BF),
            pltpu.VMEM(Wo.shape, BF),
            pltpu.VMEM((B, S, HC), BF),
            pltpu.VMEM((B, S, HC), BF),
            pltpu.VMEM((G, B, S, HC), BF),
            pltpu.SemaphoreType.DMA,
            pltpu.SemaphoreType.DMA,
            pltpu.SemaphoreType.DMA(((G - 1) * B,)),
            pltpu.SemaphoreType.DMA((G, B)),
        ],
        compiler_params=pltpu.CompilerParams(collective_id=0),
    )(x, Wdkv, Wuk, Wuv, Wq, Wqr, Wkr, Wo)


# device time: 35284 ns/iter; 1.1169x vs baseline; 1.1169x over previous
import jax
import jax.numpy as jnp
from jax import lax
from jax.experimental import pallas as pl
from jax.experimental.pallas import tpu as pltpu

H = 16
DH = 64
DR = 32
NC = 4
BF = jnp.bfloat16
F32 = jnp.float32


def _dot(a, b):
    return jnp.dot(a, b, preferred_element_type=F32)


def _dot_t(a, b):
    return lax.dot_general(a, b, (((1,), (1,)), ((), ())),
                           preferred_element_type=F32)


def kernel(x, Wdkv, Wuk, Wuv, Wq, Wqr, Wkr, Wo):
    B, S, D = x.shape
    dc = Wdkv.shape[1]
    CW = D // NC
    scale = (DH + DR) ** -0.5 * 1.4426950408889634

    def body(x_ref, wdkv_ref, wuk_ref, wuv_ref, wq_ref, wqr_ref, wkr_ref,
             wo_ref, out_ref,
             x16_ref, xmy_ref, cs_ref, csmy_ref, cr_ref,
             wukv_ref, wuvv_ref, wqv_ref, wqrv_ref, wov_ref,
             wuk16s_ref, wuv16s_ref, wukr_ref, wuvr_ref,
             q_ref, qr_ref, kr_ref, k16_ref, v32_ref, o_ref,
             outs_ref, outr_ref,
             in_sems, wsend_sems, wrecv_sems, osend_sems, orecv_sems):
        my_x = lax.axis_index("x")
        my_y = lax.axis_index("y")
        my_z = lax.axis_index("z")
        partner = (1 - my_x, my_y, my_z)

        in_dmas = [
            pltpu.make_async_copy(wuk_ref, wukv_ref, in_sems.at[0]),
            pltpu.make_async_copy(wuv_ref, wuvv_ref, in_sems.at[1]),
            pltpu.make_async_copy(wq_ref, wqv_ref, in_sems.at[2]),
            pltpu.make_async_copy(wqr_ref, wqrv_ref, in_sems.at[3]),
            pltpu.make_async_copy(wo_ref, wov_ref, in_sems.at[4]),
        ]
        for dma in in_dmas:
            dma.start()

        barrier = pltpu.get_barrier_semaphore()
        pl.semaphore_signal(barrier, inc=1, device_id=partner,
                            device_id_type=pl.DeviceIdType.MESH)
        pl.semaphore_wait(barrier, 1)

        wdkv16 = wdkv_ref[...].astype(BF)
        for b in range(B):
            x16_ref[b] = x_ref[b].astype(BF)
        for b in range(B):
            cs_ref[b] = _dot(x16_ref[b], wdkv16).astype(BF)
        c_rdma = pltpu.make_async_remote_copy(
            src_ref=cs_ref.at[1 - my_x], dst_ref=cr_ref,
            send_sem=wsend_sems.at[2], recv_sem=wrecv_sems.at[2],
            device_id=partner, device_id_type=pl.DeviceIdType.MESH)
        c_rdma.start()

        in_dmas[0].wait()
        in_dmas[1].wait()
        wuk16s_ref[...] = wukv_ref[...].astype(BF)
        wuv16s_ref[...] = wuvv_ref[...].astype(BF)
        wuk_rdma = pltpu.make_async_remote_copy(
            src_ref=wuk16s_ref, dst_ref=wukr_ref,
            send_sem=wsend_sems.at[0], recv_sem=wrecv_sems.at[0],
            device_id=partner, device_id_type=pl.DeviceIdType.MESH)
        wuk_rdma.start()
        wuv_rdma = pltpu.make_async_remote_copy(
            src_ref=wuv16s_ref, dst_ref=wuvr_ref,
            send_sem=wsend_sems.at[1], recv_sem=wrecv_sems.at[1],
            device_id=partner, device_id_type=pl.DeviceIdType.MESH)
        wuv_rdma.start()

        for bb in range(B):
            @pl.when(my_x == bb)
            def _(bb=bb):
                xmy_ref[...] = x16_ref[bb]
                csmy_ref[...] = cs_ref[bb]

        in_dmas[2].wait()
        in_dmas[3].wait()
        in_dmas[4].wait()
        wq16 = wqv_ref[...].astype(BF)
        wqr16 = wqrv_ref[...].astype(BF)
        wkr16 = wkr_ref[...].astype(BF)
        wo16 = wov_ref[...].astype(BF)
        q_ref[...] = (scale * _dot(xmy_ref[...], wq16)).astype(BF)
        qr_ref[...] = (scale * _dot(xmy_ref[...], wqr16)).astype(BF)
        kr_ref[...] = _dot(xmy_ref[...], wkr16).astype(BF)

        wuk_rdma.wait()
        wuv_rdma.wait()
        c_rdma.wait()

        wukr16 = wukr_ref[...]
        wuvr16 = wuvr_ref[...]
        k16_ref[...] = (_dot(csmy_ref[...], wuk16s_ref[...])
                        + _dot(cr_ref[...], wukr16)).astype(BF)
        v32_ref[...] = (_dot(csmy_ref[...], wuv16s_ref[...])
                        + _dot(cr_ref[...], wuvr16))

        krb = kr_ref[...]
        for h in range(H):
            qh = q_ref[:, h * DH:(h + 1) * DH]
            kh = k16_ref[:, h * DH:(h + 1) * DH]
            qrh = qr_ref[:, h * DR:(h + 1) * DR]
            s = _dot_t(qh, kh) + _dot_t(qrh, krb)
            p = jnp.exp2(s)
            denom = jnp.sum(p, axis=-1, keepdims=True)
            oh = _dot(p, v32_ref[:, h * DH:(h + 1) * DH])
            o_ref[:, h * DH:(h + 1) * DH] = (oh / denom).astype(BF)

        o_rdmas = []
        for j in range(NC):
            cols = slice(j * CW, (j + 1) * CW)
            chunk = _dot(o_ref[...], wo16[:, cols])
            for bb in range(B):
                @pl.when(my_x == bb)
                def _(bb=bb, chunk=chunk, cols=cols):
                    out_ref[bb, :, cols] = chunk
            outs_ref[:, cols] = chunk.astype(BF)
            rdma = pltpu.make_async_remote_copy(
                src_ref=outs_ref.at[:, cols], dst_ref=outr_ref.at[:, cols],
                send_sem=osend_sems.at[j], recv_sem=orecv_sems.at[j],
                device_id=partner, device_id_type=pl.DeviceIdType.MESH)
            rdma.start()
            o_rdmas.append(rdma)

        for rdma in o_rdmas:
            rdma.wait()
        for bb in range(B):
            @pl.when(my_x == bb)
            def _(bb=bb):
                out_ref[1 - bb] = outr_ref[...].astype(F32)

    return pl.pallas_call(
        body,
        out_shape=jax.ShapeDtypeStruct((B, S, D), F32),
        in_specs=[
            pl.BlockSpec(memory_space=pltpu.VMEM),
            pl.BlockSpec(memory_space=pltpu.VMEM),
            pl.BlockSpec(memory_space=pl.ANY),
            pl.BlockSpec(memory_space=pl.ANY),
            pl.BlockSpec(memory_space=pl.ANY),
            pl.BlockSpec(memory_space=pl.ANY),
            pl.BlockSpec(memory_space=pltpu.VMEM),
            pl.BlockSpec(memory_space=pl.ANY),
        ],
        out_specs=pl.BlockSpec(memory_space=pltpu.VMEM),
        scratch_shapes=[
            pltpu.VMEM((B, S, D), BF),
            pltpu.VMEM((S, D), BF),
            pltpu.VMEM((B, S, dc), BF),
            pltpu.VMEM((S, dc), BF),
            pltpu.VMEM((S, dc), BF),
            pltpu.VMEM(Wuk.shape, F32),
            pltpu.VMEM(Wuv.shape, F32),
            pltpu.VMEM(Wq.shape, F32),
            pltpu.VMEM(Wqr.shape, F32),
            pltpu.VMEM(Wo.shape, F32),
            pltpu.VMEM(Wuk.shape, BF),
            pltpu.VMEM(Wuv.shape, BF),
            pltpu.VMEM(Wuk.shape, BF),
            pltpu.VMEM(Wuv.shape, BF),
            pltpu.VMEM((S, H * DH), BF),
            pltpu.VMEM((S, H * DR), BF),
            pltpu.VMEM((S, DR), BF),
            pltpu.VMEM((S, H * DH), BF),
            pltpu.VMEM((S, H * DH), F32),
            pltpu.VMEM((S, H * DH), BF),
            pltpu.VMEM((S, D), BF),
            pltpu.VMEM((S, D), BF),
            pltpu.SemaphoreType.DMA((5,)),
            pltpu.SemaphoreType.DMA((3,)),
            pltpu.SemaphoreType.DMA((3,)),
            pltpu.SemaphoreType.DMA((NC,)),
            pltpu.SemaphoreType.DMA((NC,)),
        ],
        compiler_params=pltpu.CompilerParams(collective_id=0),
    )(x, Wdkv, Wuk, Wuv, Wq, Wqr, Wkr, Wo)
